# baseline (speedup 1.0000x reference)
import functools

import jax
import jax.numpy as jnp
from jax import lax
from jax.experimental import pallas as pl
from jax.experimental.pallas import tpu as pltpu
from jax.experimental.pallas import tpu_sc as plsc

_NC = 2
_NS = 16
_NW = _NC * _NS

_F = 16
_PK = 8
_G = 64

_CH = 128
_GRP = 8
_GE = _CH * _GRP
_NGRP = 5
_EPW = _GE * _NGRP
_EPAD = _NW * _EPW

_NROWS = 10016
_RPT = _NROWS // _NS
_NPL = _NROWS // _PK


def _sc_mesh():
    return plsc.VectorSubcoreMesh(core_axis_name="c", subcore_axis_name="s")


def _sc_gather(table_p, idx2d):
    @functools.partial(
        pl.kernel,
        mesh=_sc_mesh(),
        compiler_params=pltpu.CompilerParams(use_tc_tiling_on_sc=False),
        out_type=jax.ShapeDtypeStruct((_EPAD, _F), jnp.float32),
        scratch_types=[
            pltpu.VMEM((_NGRP * _GRP, _CH), jnp.int32),
            pltpu.VMEM((_GE, _F), jnp.float32),
            pltpu.VMEM_SHARED((_NROWS, _F), jnp.float32),
            pltpu.SemaphoreType.DMA,
        ],
    )
    def gk(tab_hbm, idx_hbm, out_hbm, idx_v, rows_v, tabs, sem):
        c = lax.axis_index("c")
        s = lax.axis_index("s")
        wid = s * _NC + c
        ebase = wid * _EPW
        rbase = wid * (_EPW // _CH)

        pltpu.sync_copy(tab_hbm.at[pl.ds(s * _RPT, _RPT)],
                        tabs.at[pl.ds(s * _RPT, _RPT)])
        pltpu.sync_copy(idx_hbm.at[pl.ds(rbase, _NGRP * _GRP)], idx_v)
        plsc.subcore_barrier()

        def body(g, carry):
            cps = [
                pltpu.async_copy(
                    tabs.at[idx_v.at[g * _GRP + j]],
                    rows_v.at[pl.ds(j * _CH, _CH)],
                    sem,
                )
                for j in range(_GRP)
            ]
            for cp in cps:
                cp.wait()
            pltpu.sync_copy(rows_v, out_hbm.at[pl.ds(ebase + g * _GE, _GE)])
            return carry

        lax.fori_loop(0, _NGRP, body, 0)

    return gk(table_p.reshape(_NROWS, _F), idx2d).reshape(_EPAD // _PK, _PK * _F)


def _sc_scatter(msg_p, dst2d, zrows, orows, half):
    epw = _EPW // 2
    grp = 4
    ge = _CH * grp
    ngrp = epw // ge
    nrow = epw // _CH
    epad_h = _EPAD // 2

    @functools.partial(
        pl.kernel,
        mesh=_sc_mesh(),
        compiler_params=pltpu.CompilerParams(use_tc_tiling_on_sc=False),
        out_type=jax.ShapeDtypeStruct((_NC, _NROWS, _F), jnp.float32),
        scratch_types=[
            pltpu.VMEM((nrow, _CH), jnp.int32),
            pltpu.VMEM((ge, _F), jnp.float32),
            pltpu.VMEM_SHARED((_NROWS, _F), jnp.float32),
            pltpu.SemaphoreType.DMA,
        ],
    )
    def sk(msg_hbm, dst_hbm, z_hbm, s_out, idx_v, rows_v, acc, sem):
        c = lax.axis_index("c")
        s = lax.axis_index("s")
        wid = s * _NC + c
        ebase = wid * epw
        rbase = wid * nrow

        pltpu.sync_copy(z_hbm, acc.at[pl.ds(s * _RPT, _RPT)])
        pltpu.sync_copy(dst_hbm.at[pl.ds(rbase, nrow)], idx_v)
        plsc.subcore_barrier()

        def body(g, carry):
            pltpu.sync_copy(msg_hbm.at[pl.ds(ebase + g * ge, ge)], rows_v)
            cps = [
                pltpu.async_copy(
                    rows_v.at[pl.ds(j * _CH, _CH)],
                    acc.at[idx_v.at[g * grp + j]],
                    sem,
                    add=True,
                )
                for j in range(grp)
            ]
            for cp in cps:
                cp.wait()
            return carry

        lax.fori_loop(0, ngrp, body, 0)
        plsc.subcore_barrier()

        pltpu.sync_copy(
            acc.at[pl.ds(s * _RPT, _RPT)],
            s_out.at[c, pl.ds(s * _RPT, _RPT)],
        )

    res = sk(msg_p.reshape(epad_h, _F), dst2d, zrows)
    return res.reshape(_NC, _NPL, _PK * _F)


def _sc_count(dst2d, zrows, orows):
    @functools.partial(
        pl.kernel,
        mesh=_sc_mesh(),
        compiler_params=pltpu.CompilerParams(use_tc_tiling_on_sc=False),
        out_type=jax.ShapeDtypeStruct((_NC, _NROWS, _F), jnp.float32),
        scratch_types=[
            pltpu.VMEM((_NGRP * _GRP, _CH), jnp.int32),
            pltpu.VMEM((_CH, _F), jnp.float32),
            pltpu.VMEM_SHARED((_NROWS, _F), jnp.float32),
            pltpu.SemaphoreType.DMA,
        ],
    )
    def ck(dst_hbm, z_hbm, o_hbm, c_out, idx_v, one_v, cacc, sem):
        c = lax.axis_index("c")
        s = lax.axis_index("s")
        wid = s * _NC + c
        rbase = wid * (_EPW // _CH)

        pltpu.sync_copy(z_hbm, cacc.at[pl.ds(s * _RPT, _RPT)])
        pltpu.sync_copy(dst_hbm.at[pl.ds(rbase, _NGRP * _GRP)], idx_v)
        pltpu.sync_copy(o_hbm, one_v)
        plsc.subcore_barrier()

        def body(g, carry):
            cps = [
                pltpu.async_copy(
                    one_v,
                    cacc.at[idx_v.at[g * _GRP + j]],
                    sem,
                    add=True,
                )
                for j in range(_GRP)
            ]
            for cp in cps:
                cp.wait()
            return carry

        lax.fori_loop(0, _NGRP, body, 0)
        plsc.subcore_barrier()

        pltpu.sync_copy(
            cacc.at[pl.ds(s * _RPT, _RPT)],
            c_out.at[c, pl.ds(s * _RPT, _RPT)],
        )

    return _sc_count_post(ck(dst2d, zrows, orows))


def _sc_count_post(r):
    return r.reshape(_NC, _NPL, _PK * _F)


_BR = 2048
_BE = _BR * _PK


def _tc_msg(ea_t, xsrc_p, w1a, w2, b2r, tmat, smat):
    nb = xsrc_p.shape[0] * _PK // _BE

    def body(ea_ref, xs_ref, w1_ref, w2_ref, b2_ref, t_ref, s_ref,
             out_ref):
        ea_b = jnp.transpose(ea_ref[...])
        h = jnp.maximum(
            jnp.dot(ea_b, w1_ref[...], preferred_element_type=jnp.float32),
            0.0,
        )
        w = jnp.dot(h, w2_ref[...], preferred_element_type=jnp.float32) \
            + b2_ref[...]
        xs128 = xs_ref[...]
        outs = []
        for k in range(_PK):
            xs_k = xs128[:, k * _F:(k + 1) * _F]
            xjt = jnp.dot(xs_k, t_ref[...],
                          preferred_element_type=jnp.float32)
            p_k = w[k * _BR:(k + 1) * _BR, :] * xjt
            outs.append(jnp.dot(p_k.astype(jnp.bfloat16), s_ref[...],
                                preferred_element_type=jnp.float32))
        out_ref[...] = jnp.concatenate(outs, axis=1)

    full = lambda shape: pl.BlockSpec(shape, lambda i: (0, 0))
    return pl.pallas_call(
        body,
        grid=(nb,),
        in_specs=[
            pl.BlockSpec((3, _BE), lambda i: (0, i)),
            pl.BlockSpec((_BR, _PK * _F), lambda i: (i, 0)),
            full((3, 32)),
            full((32, _F * _F)),
            full((1, _F * _F)),
            full((_F, _F * _F)),
            full((_F * _F, _F)),
        ],
        out_specs=pl.BlockSpec((_BR, _PK * _F), lambda i: (i, 0)),
        out_shape=jax.ShapeDtypeStruct((xsrc_p.shape[0], _PK * _F),
                                       jnp.float32),
    )(ea_t, xsrc_p, w1a, w2, b2r, tmat, smat)


def _tc_update_first(spa, spb, cp, fp, r128, b128):
    def body(spa_ref, spb_ref, cp_ref, f_ref, r_ref, b_ref, out_ref, ic_ref):
        cnt = cp_ref[0] + cp_ref[1]
        ic = 1.0 / jnp.maximum(cnt, 1.0)
        ic_ref[...] = ic
        ssum = (spa_ref[0] + spa_ref[1]) + (spb_ref[0] + spb_ref[1])
        upd = ssum * ic + jnp.dot(f_ref[...], r_ref[...],
                                  preferred_element_type=jnp.float32) \
            + b_ref[...]
        out_ref[...] = jnp.maximum(upd, 0.0)

    return pl.pallas_call(
        body,
        out_shape=[
            jax.ShapeDtypeStruct((_NPL, _PK * _F), jnp.float32),
            jax.ShapeDtypeStruct((_NPL, _PK * _F), jnp.float32),
        ],
    )(spa, spb, cp, fp, r128, b128)


def _tc_update(spa, spb, ic, fp, r128, b128):
    def body(spa_ref, spb_ref, ic_ref, f_ref, r_ref, b_ref, out_ref):
        ssum = (spa_ref[0] + spa_ref[1]) + (spb_ref[0] + spb_ref[1])
        upd = ssum * ic_ref[...] + jnp.dot(f_ref[...], r_ref[...],
                                           preferred_element_type=jnp.float32) \
            + b_ref[...]
        out_ref[...] = jnp.maximum(upd, 0.0)

    return pl.pallas_call(
        body,
        out_shape=jax.ShapeDtypeStruct((_NPL, _PK * _F), jnp.float32),
    )(spa, spb, ic, fp, r128, b128)


def _tc_update_pool(spa, spb, ic, fp, r128, b128, batch_t):
    def body(spa_ref, spb_ref, ic_ref, f_ref, r_ref, b_ref, batch_ref, out_ref):
        ssum = (spa_ref[0] + spa_ref[1]) + (spb_ref[0] + spb_ref[1])
        upd = ssum * ic_ref[...] + jnp.dot(f_ref[...], r_ref[...],
                                           preferred_element_type=jnp.float32) \
            + b_ref[...]
        h3 = jnp.maximum(upd, 0.0)
        gids = lax.broadcasted_iota(jnp.int32, (_G, _NPL), 0)
        pool_s = jnp.zeros((_G, _F), jnp.float32)
        cnt = jnp.zeros((_G, 1), jnp.float32)
        for k in range(_PK):
            onehot = (gids == batch_ref[k:k + 1, :]).astype(jnp.float32)
            pool_s = pool_s + jnp.dot(onehot, h3[:, k * _F:(k + 1) * _F],
                                      preferred_element_type=jnp.float32)
            cnt = cnt + jnp.sum(onehot, axis=1, keepdims=True)
        out_ref[...] = pool_s / jnp.maximum(cnt, 1.0)

    return pl.pallas_call(
        body,
        out_shape=jax.ShapeDtypeStruct((_G, _F), jnp.float32),
    )(spa, spb, ic, fp, r128, b128, batch_t)


def kernel(x, edge_index, edge_attr, batch,
           w1_1, b1_1, w2_1, b2_1, root1, bias1,
           w1_2, b1_2, w2_2, b2_2, root2, bias2,
           w1_3, b1_3, w2_3, b2_3, root3, bias3):
    n = x.shape[0]
    e = edge_index.shape[1]
    pad = _EPAD - e
    nb = _EPAD // _BE

    src_p = jnp.concatenate(
        [edge_index[0], jnp.zeros((pad,), jnp.int32)]).reshape(-1, _CH)
    dst_p = jnp.concatenate(
        [edge_index[1], jnp.full((pad,), n, jnp.int32)]).reshape(-1, _CH)
    ea01 = jnp.pad(jnp.transpose(edge_attr), ((0, 0), (0, pad))) \
        .reshape(2, nb, _BR, _PK).transpose(0, 1, 3, 2).reshape(2, _EPAD)
    ea_t = jnp.concatenate([ea01, jnp.ones((1, _EPAD), jnp.float32)])

    x_p = jnp.concatenate(
        [x, jnp.zeros((_NROWS - n, _F), jnp.float32)]).reshape(_NPL, _PK * _F)
    batch_t = jnp.concatenate(
        [batch.astype(jnp.int32),
         jnp.full((_NROWS - n,), jnp.int32(2 ** 30), jnp.int32)]) \
        .reshape(_NPL, _PK).transpose(1, 0)

    tmat = jnp.repeat(jnp.eye(_F, dtype=jnp.float32), _F, axis=1)
    smat = jnp.tile(jnp.eye(_F, dtype=jnp.bfloat16), (_F, 1))
    zrows = jnp.zeros((_RPT, _F), jnp.float32)
    orows = jnp.ones((_CH, _F), jnp.float32)
    eye8 = jnp.eye(_PK, dtype=jnp.float32)

    w1s = (w1_1, w1_2, w1_3)
    b1s = (b1_1, b1_2, b1_3)
    w2s = (w2_1, w2_2, w2_3)
    b2s = (b2_1, b2_2, b2_3)
    roots = (root1, root2, root3)
    biases = (bias1, bias2, bias3)

    fp = x_p
    ic = None
    out = None
    cp = _sc_count(dst_p, zrows, orows)
    for layer in range(3):
        r128 = jnp.kron(eye8, roots[layer])
        b128 = jnp.tile(biases[layer], _PK).reshape(1, _PK * _F)
        w1a = jnp.concatenate([w1s[layer], b1s[layer].reshape(1, -1)], axis=0)
        xsrc = _sc_gather(fp, src_p)
        hpl = _EPAD // _PK // 2
        hco = _EPAD // 2
        hdr = _EPAD // _CH // 2
        b2r = b2s[layer].reshape(1, -1)
        sps = []
        for hf in range(2):
            msg_h = _tc_msg(ea_t[:, hf * hco:(hf + 1) * hco],
                            xsrc[hf * hpl:(hf + 1) * hpl],
                            w1a, w2s[layer], b2r, tmat, smat)
            sps.append(_sc_scatter(msg_h, dst_p[hf * hdr:(hf + 1) * hdr],
                                   zrows, orows, hf))
        if layer == 0:
            fp, ic = _tc_update_first(sps[0], sps[1], cp, fp, r128, b128)
        elif layer == 1:
            fp = _tc_update(sps[0], sps[1], ic, fp, r128, b128)
        else:
            out = _tc_update_pool(sps[0], sps[1], ic, fp, r128, b128, batch_t)
    return out


# speedup vs baseline: 1.7895x; 1.7895x over previous
import functools

import jax
import jax.numpy as jnp
from jax import lax
from jax.experimental import pallas as pl
from jax.experimental.pallas import tpu as pltpu
from jax.experimental.pallas import tpu_sc as plsc

_NC = 2
_NS = 16
_NW = _NC * _NS

_F = 16
_PK = 8
_G = 64

_CH = 128
_GRP = 8
_GE = _CH * _GRP
_NGRP = 5
_EPW = _GE * _NGRP
_EPAD = _NW * _EPW

_NROWS = 10016
_RPT = _NROWS // _NS
_NPL = _NROWS // _PK


def _sc_mesh():
    return plsc.VectorSubcoreMesh(core_axis_name="c", subcore_axis_name="s")


def _sc_gather(table_p, idx2d):
    @functools.partial(
        pl.kernel,
        mesh=_sc_mesh(),
        compiler_params=pltpu.CompilerParams(use_tc_tiling_on_sc=False),
        out_type=jax.ShapeDtypeStruct((_EPAD, _F), jnp.float32),
        scratch_types=[
            pltpu.VMEM((_NGRP * _GRP, _CH), jnp.int32),
            pltpu.VMEM((_GE, _F), jnp.float32),
            pltpu.VMEM_SHARED((_NROWS, _F), jnp.float32),
            pltpu.SemaphoreType.DMA,
        ],
    )
    def gk(tab_hbm, idx_hbm, out_hbm, idx_v, rows_v, tabs, sem):
        c = lax.axis_index("c")
        s = lax.axis_index("s")
        wid = s * _NC + c
        ebase = wid * _EPW
        rbase = wid * (_EPW // _CH)

        pltpu.sync_copy(tab_hbm.at[pl.ds(s * _RPT, _RPT)],
                        tabs.at[pl.ds(s * _RPT, _RPT)])
        pltpu.sync_copy(idx_hbm.at[pl.ds(rbase, _NGRP * _GRP)], idx_v)
        plsc.subcore_barrier()

        def body(g, carry):
            cps = [
                pltpu.async_copy(
                    tabs.at[idx_v.at[g * _GRP + j]],
                    rows_v.at[pl.ds(j * _CH, _CH)],
                    sem,
                )
                for j in range(_GRP)
            ]
            for cp in cps:
                cp.wait()
            pltpu.sync_copy(rows_v, out_hbm.at[pl.ds(ebase + g * _GE, _GE)])
            return carry

        lax.fori_loop(0, _NGRP, body, 0)

    return gk(table_p.reshape(_NROWS, _F), idx2d).reshape(_EPAD // _PK, _PK * _F)


def _sc_scatter(msg_p, dst2d, zrows):
    @functools.partial(
        pl.kernel,
        mesh=_sc_mesh(),
        compiler_params=pltpu.CompilerParams(use_tc_tiling_on_sc=False),
        out_type=jax.ShapeDtypeStruct((_NC, _NROWS, _F), jnp.float32),
        scratch_types=[
            pltpu.VMEM((_NGRP * _GRP, _CH), jnp.int32),
            pltpu.VMEM((_GE, _F), jnp.float32),
            pltpu.VMEM_SHARED((_NROWS, _F), jnp.float32),
            pltpu.SemaphoreType.DMA,
        ],
    )
    def sk(msg_hbm, dst_hbm, z_hbm, s_out, idx_v, rows_v, acc, sem):
        c = lax.axis_index("c")
        s = lax.axis_index("s")
        wid = s * _NC + c
        ebase = wid * _EPW
        rbase = wid * (_EPW // _CH)

        pltpu.sync_copy(z_hbm, acc.at[pl.ds(s * _RPT, _RPT)])
        pltpu.sync_copy(dst_hbm.at[pl.ds(rbase, _NGRP * _GRP)], idx_v)
        plsc.subcore_barrier()

        def body(g, carry):
            pltpu.sync_copy(msg_hbm.at[pl.ds(ebase + g * _GE, _GE)], rows_v)
            cps = [
                pltpu.async_copy(
                    rows_v.at[pl.ds(j * _CH, _CH)],
                    acc.at[idx_v.at[g * _GRP + j]],
                    sem,
                    add=True,
                )
                for j in range(_GRP)
            ]
            for cp in cps:
                cp.wait()
            return carry

        lax.fori_loop(0, _NGRP, body, 0)
        plsc.subcore_barrier()

        pltpu.sync_copy(
            acc.at[pl.ds(s * _RPT, _RPT)],
            s_out.at[c, pl.ds(s * _RPT, _RPT)],
        )

    res = sk(msg_p.reshape(_EPAD, _F), dst2d, zrows)
    return res.reshape(_NC, _NPL, _PK * _F)


def _sc_count(dst2d, zrows, orows):
    @functools.partial(
        pl.kernel,
        mesh=_sc_mesh(),
        compiler_params=pltpu.CompilerParams(use_tc_tiling_on_sc=False),
        out_type=jax.ShapeDtypeStruct((_NC, _NROWS, _F), jnp.float32),
        scratch_types=[
            pltpu.VMEM((_NGRP * _GRP, _CH), jnp.int32),
            pltpu.VMEM((_CH, _F), jnp.float32),
            pltpu.VMEM_SHARED((_NROWS, _F), jnp.float32),
            pltpu.SemaphoreType.DMA,
        ],
    )
    def ck(dst_hbm, z_hbm, o_hbm, c_out, idx_v, one_v, cacc, sem):
        c = lax.axis_index("c")
        s = lax.axis_index("s")
        wid = s * _NC + c
        rbase = wid * (_EPW // _CH)

        pltpu.sync_copy(z_hbm, cacc.at[pl.ds(s * _RPT, _RPT)])
        pltpu.sync_copy(dst_hbm.at[pl.ds(rbase, _NGRP * _GRP)], idx_v)
        pltpu.sync_copy(o_hbm, one_v)
        plsc.subcore_barrier()

        def body(g, carry):
            cps = [
                pltpu.async_copy(
                    one_v,
                    cacc.at[idx_v.at[g * _GRP + j]],
                    sem,
                    add=True,
                )
                for j in range(_GRP)
            ]
            for cp in cps:
                cp.wait()
            return carry

        lax.fori_loop(0, _NGRP, body, 0)
        plsc.subcore_barrier()

        pltpu.sync_copy(
            cacc.at[pl.ds(s * _RPT, _RPT)],
            c_out.at[c, pl.ds(s * _RPT, _RPT)],
        )

    return _sc_count_post(ck(dst2d, zrows, orows))


def _sc_count_post(r):
    return r.reshape(_NC, _NPL, _PK * _F)


_BR = 2048
_BE = _BR * _PK


def _tc_msg(ea_t, xsrc_p, w1a, w2, b2r, tmat, smat):
    nb = xsrc_p.shape[0] * _PK // _BE

    def body(ea_ref, xs_ref, w1_ref, w2_ref, b2_ref, t_ref, s_ref,
             out_ref):
        ea_b = jnp.transpose(ea_ref[...])
        h = jnp.maximum(
            jnp.dot(ea_b, w1_ref[...], preferred_element_type=jnp.float32),
            0.0,
        )
        w = jnp.dot(h, w2_ref[...], preferred_element_type=jnp.float32) \
            + b2_ref[...]
        xs128 = xs_ref[...]
        outs = []
        for k in range(_PK):
            xs_k = xs128[:, k * _F:(k + 1) * _F]
            xjt = jnp.dot(xs_k, t_ref[...],
                          preferred_element_type=jnp.float32)
            p_k = w[k * _BR:(k + 1) * _BR, :] * xjt
            outs.append(jnp.dot(p_k.astype(jnp.bfloat16), s_ref[...],
                                preferred_element_type=jnp.float32))
        out_ref[...] = jnp.concatenate(outs, axis=1)

    full = lambda shape: pl.BlockSpec(shape, lambda i: (0, 0))
    return pl.pallas_call(
        body,
        grid=(nb,),
        in_specs=[
            pl.BlockSpec((3, _BE), lambda i: (0, i)),
            pl.BlockSpec((_BR, _PK * _F), lambda i: (i, 0)),
            full((3, 32)),
            full((32, _F * _F)),
            full((1, _F * _F)),
            full((_F, _F * _F)),
            full((_F * _F, _F)),
        ],
        out_specs=pl.BlockSpec((_BR, _PK * _F), lambda i: (i, 0)),
        out_shape=jax.ShapeDtypeStruct((xsrc_p.shape[0], _PK * _F),
                                       jnp.float32),
    )(ea_t, xsrc_p, w1a, w2, b2r, tmat, smat)


def _tc_update_first(sp, cp, fp, r128, b128):
    def body(sp_ref, cp_ref, f_ref, r_ref, b_ref, out_ref, ic_ref):
        cnt = cp_ref[0] + cp_ref[1]
        ic = 1.0 / jnp.maximum(cnt, 1.0)
        ic_ref[...] = ic
        ssum = sp_ref[0] + sp_ref[1]
        upd = ssum * ic + jnp.dot(f_ref[...], r_ref[...],
                                  preferred_element_type=jnp.float32) \
            + b_ref[...]
        out_ref[...] = jnp.maximum(upd, 0.0)

    return pl.pallas_call(
        body,
        out_shape=[
            jax.ShapeDtypeStruct((_NPL, _PK * _F), jnp.float32),
            jax.ShapeDtypeStruct((_NPL, _PK * _F), jnp.float32),
        ],
    )(sp, cp, fp, r128, b128)


def _tc_update(sp, ic, fp, r128, b128):
    def body(sp_ref, ic_ref, f_ref, r_ref, b_ref, out_ref):
        ssum = sp_ref[0] + sp_ref[1]
        upd = ssum * ic_ref[...] + jnp.dot(f_ref[...], r_ref[...],
                                           preferred_element_type=jnp.float32) \
            + b_ref[...]
        out_ref[...] = jnp.maximum(upd, 0.0)

    return pl.pallas_call(
        body,
        out_shape=jax.ShapeDtypeStruct((_NPL, _PK * _F), jnp.float32),
    )(sp, ic, fp, r128, b128)


def _tc_update_pool(sp, ic, fp, r128, b128, batch_t):
    def body(sp_ref, ic_ref, f_ref, r_ref, b_ref, batch_ref, out_ref):
        ssum = sp_ref[0] + sp_ref[1]
        upd = ssum * ic_ref[...] + jnp.dot(f_ref[...], r_ref[...],
                                           preferred_element_type=jnp.float32) \
            + b_ref[...]
        h3 = jnp.maximum(upd, 0.0)
        gids = lax.broadcasted_iota(jnp.int32, (_G, _NPL), 0)
        pool_s = jnp.zeros((_G, _F), jnp.float32)
        cnt = jnp.zeros((_G, 1), jnp.float32)
        for k in range(_PK):
            onehot = (gids == batch_ref[k:k + 1, :]).astype(jnp.float32)
            pool_s = pool_s + jnp.dot(onehot, h3[:, k * _F:(k + 1) * _F],
                                      preferred_element_type=jnp.float32)
            cnt = cnt + jnp.sum(onehot, axis=1, keepdims=True)
        out_ref[...] = pool_s / jnp.maximum(cnt, 1.0)

    return pl.pallas_call(
        body,
        out_shape=jax.ShapeDtypeStruct((_G, _F), jnp.float32),
    )(sp, ic, fp, r128, b128, batch_t)


def kernel(x, edge_index, edge_attr, batch,
           w1_1, b1_1, w2_1, b2_1, root1, bias1,
           w1_2, b1_2, w2_2, b2_2, root2, bias2,
           w1_3, b1_3, w2_3, b2_3, root3, bias3):
    n = x.shape[0]
    e = edge_index.shape[1]
    pad = _EPAD - e
    nb = _EPAD // _BE

    src_p = jnp.concatenate(
        [edge_index[0], jnp.zeros((pad,), jnp.int32)]).reshape(-1, _CH)
    dst_p = jnp.concatenate(
        [edge_index[1], jnp.full((pad,), n, jnp.int32)]).reshape(-1, _CH)
    ea01 = jnp.pad(jnp.transpose(edge_attr), ((0, 0), (0, pad))) \
        .reshape(2, nb, _BR, _PK).transpose(0, 1, 3, 2).reshape(2, _EPAD)
    ea_t = jnp.concatenate([ea01, jnp.ones((1, _EPAD), jnp.float32)])

    x_p = jnp.concatenate(
        [x, jnp.zeros((_NROWS - n, _F), jnp.float32)]).reshape(_NPL, _PK * _F)
    batch_t = jnp.concatenate(
        [batch.astype(jnp.int32),
         jnp.full((_NROWS - n,), jnp.int32(2 ** 30), jnp.int32)]) \
        .reshape(_NPL, _PK).transpose(1, 0)

    tmat = jnp.repeat(jnp.eye(_F, dtype=jnp.float32), _F, axis=1)
    smat = jnp.tile(jnp.eye(_F, dtype=jnp.bfloat16), (_F, 1))
    zrows = jnp.zeros((_RPT, _F), jnp.float32)
    orows = jnp.ones((_CH, _F), jnp.float32)
    eye8 = jnp.eye(_PK, dtype=jnp.float32)

    w1s = (w1_1, w1_2, w1_3)
    b1s = (b1_1, b1_2, b1_3)
    w2s = (w2_1, w2_2, w2_3)
    b2s = (b2_1, b2_2, b2_3)
    roots = (root1, root2, root3)
    biases = (bias1, bias2, bias3)

    fp = x_p
    ic = None
    out = None
    cp = _sc_count(dst_p, zrows, orows)
    for layer in range(3):
        r128 = jnp.kron(eye8, roots[layer])
        b128 = jnp.tile(biases[layer], _PK).reshape(1, _PK * _F)
        w1a = jnp.concatenate([w1s[layer], b1s[layer].reshape(1, -1)], axis=0)
        xsrc = _sc_gather(fp, src_p)
        msg = _tc_msg(ea_t, xsrc, w1a, w2s[layer],
                      b2s[layer].reshape(1, -1), tmat, smat)
        sp = _sc_scatter(msg, dst_p, zrows)
        if layer == 0:
            fp, ic = _tc_update_first(sp, cp, fp, r128, b128)
        elif layer == 1:
            fp = _tc_update(sp, ic, fp, r128, b128)
        else:
            out = _tc_update_pool(sp, ic, fp, r128, b128, batch_t)
    return out
